# baseline (speedup 1.0000x reference)
import functools

import jax
import jax.numpy as jnp
from jax import lax
from jax.experimental import pallas as pl
from jax.experimental.pallas import tpu as pltpu
from jax.experimental.pallas import tpu_sc as plsc

R = 2.0
T = 1.0

NC = 2
NS = 16
NW = NC * NS
L = 16

N_NODES = 10000
D = 128
N_EDGES = 320000
E_W = N_EDGES // NW
CHUNK = 128
NCH = E_W // CHUNK
TAIL = E_W - NCH * CHUNK
NBUF = 4


def _renorm_tc(h):
    blk = 1000

    def body(h_ref, o_ref, on_ref):
        x = h_ref[...]
        ss = jnp.sum(x * x, axis=1, keepdims=True)
        norm = jnp.sqrt(ss)
        scale = jnp.where(norm > 1.0, 1.0 / jnp.maximum(norm, 1e-12), 1.0)
        y = x * scale
        o_ref[...] = y
        on_ref[...] = -y

    return pl.pallas_call(
        body,
        out_shape=[jax.ShapeDtypeStruct((N_NODES, D), jnp.float32),
                   jax.ShapeDtypeStruct((N_NODES, D), jnp.float32)],
        grid=(N_NODES // blk,),
        in_specs=[pl.BlockSpec((blk, D), lambda i: (i, 0))],
        out_specs=[pl.BlockSpec((blk, D), lambda i: (i, 0)),
                   pl.BlockSpec((blk, D), lambda i: (i, 0))],
    )(h)


def _decode_sc(tab, tabn, idx0, idx1):
    mesh = plsc.VectorSubcoreMesh(core_axis_name="c", subcore_axis_name="s")

    @functools.partial(
        pl.kernel,
        out_type=jax.ShapeDtypeStruct((N_EDGES,), jnp.float32),
        mesh=mesh,
        scratch_types=[
            pltpu.VMEM((E_W,), jnp.int32),
            pltpu.VMEM((E_W,), jnp.int32),
            pltpu.VMEM((NBUF, CHUNK, D), jnp.float32),
            pltpu.VMEM((E_W,), jnp.float32),
            pltpu.SemaphoreType.DMA,
            pltpu.SemaphoreType.DMA,
            pltpu.SemaphoreType.DMA,
            pltpu.SemaphoreType.DMA,
            pltpu.SemaphoreType.DMA,
            pltpu.SemaphoreType.DMA,
            pltpu.SemaphoreType.DMA,
            pltpu.SemaphoreType.DMA,
        ],
    )
    def decode(tab_hbm, tabn_hbm, idx0_hbm, idx1_hbm, out_hbm,
               idx0_all, idx1_all, rows, out_all,
               semp0, semp1, semp2, semp3, sema0, sema1, sema2, sema3):
        semp = [semp0, semp1, semp2, semp3]
        sema = [sema0, sema1, sema2, sema3]
        wid = lax.axis_index("s") * NC + lax.axis_index("c")
        base = wid * E_W

        pltpu.sync_copy(idx0_hbm.at[pl.ds(base, E_W)], idx0_all)
        pltpu.sync_copy(idx1_hbm.at[pl.ds(base, E_W)], idx1_all)

        def start_plain(ci, b):
            off = ci * CHUNK
            pltpu.async_copy(tab_hbm.at[idx0_all.at[pl.ds(off, CHUNK)]],
                             rows.at[b], semp[b])

        def wait_plain(b):
            pltpu.make_async_copy(tab_hbm.at[pl.ds(0, CHUNK)],
                                  rows.at[b], semp[b]).wait()

        def start_add(ci, b):
            off = ci * CHUNK
            pltpu.async_copy(tabn_hbm.at[idx1_all.at[pl.ds(off, CHUNK)]],
                             rows.at[b], sema[b], add=True)

        def wait_add(b):
            pltpu.make_async_copy(tabn_hbm.at[pl.ds(0, CHUNK)],
                                  rows.at[b], sema[b]).wait()

        lane = lax.iota(jnp.int32, L)
        dnums = lax.GatherDimensionNumbers(
            offset_dims=(), collapsed_slice_dims=(0,),
            start_index_map=(0,))

        def xperm(v, p):
            return lax.gather(v, p, dnums, slice_sizes=(1,),
                              mode=lax.GatherScatterMode.PROMISE_IN_BOUNDS)

        tree_perm = [(lane ^ (1 << k))[:, None] for k in range(4)]
        tree_mask = [(lane & (1 << k)) == 0 for k in range(4)]

        def merge_level(vs, k):
            p, m = tree_perm[k], tree_mask[k]
            return [jnp.where(m, x + xperm(x, p), y + xperm(y, p))
                    for x, y in zip(vs[0::2], vs[1::2])]

        def group_probs(r, g):
            quads = []
            vs = []
            for k in range(L):
                e = g * L + k
                acc = None
                for d in range(D // L):
                    df = r[e, pl.ds(d * L, L)]
                    sq = df * df
                    acc = sq if acc is None else acc + sq
                vs.append(acc)
                if len(vs) == 4:
                    quads.append(merge_level(merge_level(vs, 0), 1)[0])
                    vs = []
            res = merge_level(merge_level(quads, 2), 3)[0]
            return 1.0 / (jnp.exp((res - R) / T) + 1.0)

        def compute(ci, b):
            obase = ci * CHUNK
            r = rows.at[b]

            def group_body(g, c2):
                out_all[pl.ds(obase + g * L, L)] = group_probs(r, g)
                return c2

            lax.fori_loop(0, CHUNK // L, group_body, 0, unroll=False)

        start_plain(0, 0)
        start_plain(1, 1)
        start_plain(2, 2)
        wait_plain(0)
        start_add(0, 0)

        def ring_body(gg, carry):
            for b in range(NBUF):
                ci = NBUF * gg + b

                @pl.when(ci + 3 < NCH)
                def _():
                    start_plain(ci + 3, (b + 3) % NBUF)

                @pl.when(ci + 1 < NCH)
                def _():
                    wait_plain((b + 1) % NBUF)
                    start_add(ci + 1, (b + 1) % NBUF)

                wait_add(b)
                compute(ci, b)

            return carry

        lax.fori_loop(0, NCH // NBUF, ring_body, 0, unroll=False)

        for ci in range(NCH - NCH % NBUF, NCH):
            b = ci % NBUF

            @pl.when(ci + 3 < NCH)
            def _():
                start_plain(ci + 3, (b + 3) % NBUF)

            @pl.when(ci + 1 < NCH)
            def _():
                wait_plain((b + 1) % NBUF)
                start_add(ci + 1, (b + 1) % NBUF)

            wait_add(b)
            compute(ci, b)

        if TAIL:
            toff = NCH * CHUNK
            tb = rows.at[0].at[pl.ds(0, TAIL)]
            pltpu.async_copy(tab_hbm.at[idx0_all.at[pl.ds(toff, TAIL)]],
                             tb, semp[0])
            pltpu.make_async_copy(tab_hbm.at[pl.ds(0, TAIL)], tb,
                                  semp[0]).wait()
            pltpu.async_copy(tabn_hbm.at[idx1_all.at[pl.ds(toff, TAIL)]],
                             tb, sema[0], add=True)
            pltpu.make_async_copy(tabn_hbm.at[pl.ds(0, TAIL)], tb,
                                  sema[0]).wait()
            r = rows.at[0]
            for g in range(TAIL // L):
                out_all[pl.ds(toff + g * L, L)] = group_probs(r, g)

        pltpu.sync_copy(out_all, out_hbm.at[pl.ds(base, E_W)])

    return decode(tab, tabn, idx0, idx1)


def kernel(h, idx):
    idx = idx.astype(jnp.int32)
    idx0 = idx[:, 0]
    idx1 = idx[:, 1]
    tab, tabn = _renorm_tc(h)
    return _decode_sc(tab, tabn, idx0, idx1)


# speedup vs baseline: 1.1196x; 1.1196x over previous
import functools

import jax
import jax.numpy as jnp
from jax import lax
from jax.experimental import pallas as pl
from jax.experimental.pallas import tpu as pltpu
from jax.experimental.pallas import tpu_sc as plsc

R = 2.0
T = 1.0

NC = 2
NS = 16
NW = NC * NS
L = 16

N_NODES = 10000
D = 128
N_EDGES = 320000
E_W = N_EDGES // NW
CHUNK = 128
NCH = E_W // CHUNK
TAIL = E_W - NCH * CHUNK
NBUF = 3


def _renorm_tc(h):
    blk = 1000

    def body(h_ref, o_ref, on_ref):
        x = h_ref[...]
        ss = jnp.sum(x * x, axis=1, keepdims=True)
        norm = jnp.sqrt(ss)
        scale = jnp.where(norm > 1.0, 1.0 / jnp.maximum(norm, 1e-12), 1.0)
        y = x * scale
        o_ref[...] = y
        on_ref[...] = -y

    return pl.pallas_call(
        body,
        out_shape=[jax.ShapeDtypeStruct((N_NODES, D), jnp.float32),
                   jax.ShapeDtypeStruct((N_NODES, D), jnp.float32)],
        grid=(N_NODES // blk,),
        in_specs=[pl.BlockSpec((blk, D), lambda i: (i, 0))],
        out_specs=[pl.BlockSpec((blk, D), lambda i: (i, 0)),
                   pl.BlockSpec((blk, D), lambda i: (i, 0))],
    )(h)


def _decode_sc(tab, tabn, idx0, idx1):
    mesh = plsc.VectorSubcoreMesh(core_axis_name="c", subcore_axis_name="s")

    @functools.partial(
        pl.kernel,
        out_type=jax.ShapeDtypeStruct((N_EDGES,), jnp.float32),
        mesh=mesh,
        scratch_types=[
            pltpu.VMEM((E_W,), jnp.int32),
            pltpu.VMEM((E_W,), jnp.int32),
            pltpu.VMEM((NBUF, CHUNK, D), jnp.float32),
            pltpu.VMEM((E_W,), jnp.float32),
            pltpu.SemaphoreType.DMA,
            pltpu.SemaphoreType.DMA,
            pltpu.SemaphoreType.DMA,
            pltpu.SemaphoreType.DMA,
            pltpu.SemaphoreType.DMA,
            pltpu.SemaphoreType.DMA,
        ],
    )
    def decode(tab_hbm, tabn_hbm, idx0_hbm, idx1_hbm, out_hbm,
               idx0_all, idx1_all, rows, out_all,
               semp0, semp1, semp2, sema0, sema1, sema2):
        semp = [semp0, semp1, semp2]
        sema = [sema0, sema1, sema2]
        wid = lax.axis_index("s") * NC + lax.axis_index("c")
        base = wid * E_W

        pltpu.sync_copy(idx0_hbm.at[pl.ds(base, E_W)], idx0_all)
        pltpu.sync_copy(idx1_hbm.at[pl.ds(base, E_W)], idx1_all)

        def start_plain(ci, b):
            off = ci * CHUNK
            pltpu.async_copy(tab_hbm.at[idx0_all.at[pl.ds(off, CHUNK)]],
                             rows.at[b], semp[b])

        def wait_plain(b):
            pltpu.make_async_copy(tab_hbm.at[pl.ds(0, CHUNK)],
                                  rows.at[b], semp[b]).wait()

        def start_add(ci, b):
            off = ci * CHUNK
            pltpu.async_copy(tabn_hbm.at[idx1_all.at[pl.ds(off, CHUNK)]],
                             rows.at[b], sema[b], add=True)

        def wait_add(b):
            pltpu.make_async_copy(tabn_hbm.at[pl.ds(0, CHUNK)],
                                  rows.at[b], sema[b]).wait()

        lane = lax.iota(jnp.int32, L)
        dnums = lax.GatherDimensionNumbers(
            offset_dims=(), collapsed_slice_dims=(0,),
            start_index_map=(0,))

        def xperm(v, p):
            return lax.gather(v, p, dnums, slice_sizes=(1,),
                              mode=lax.GatherScatterMode.PROMISE_IN_BOUNDS)

        tree_perm = [(lane ^ (1 << k))[:, None] for k in range(4)]
        tree_mask = [(lane & (1 << k)) == 0 for k in range(4)]

        def merge_level(vs, k):
            p, m = tree_perm[k], tree_mask[k]
            return [jnp.where(m, x + xperm(x, p), y + xperm(y, p))
                    for x, y in zip(vs[0::2], vs[1::2])]

        def group_probs(r, g):
            quads = []
            vs = []
            for k in range(L):
                e = g * L + k
                acc = None
                for d in range(D // L):
                    df = r[e, pl.ds(d * L, L)]
                    sq = df * df
                    acc = sq if acc is None else acc + sq
                vs.append(acc)
                if len(vs) == 4:
                    quads.append(merge_level(merge_level(vs, 0), 1)[0])
                    vs = []
            res = merge_level(merge_level(quads, 2), 3)[0]
            return 1.0 / (jnp.exp((res - R) / T) + 1.0)

        def compute(ci, b):
            obase = ci * CHUNK
            r = rows.at[b]

            def group_body(g, c2):
                out_all[pl.ds(obase + g * L, L)] = group_probs(r, g)
                return c2

            lax.fori_loop(0, CHUNK // L, group_body, 0, unroll=False)

        start_plain(0, 0)
        start_plain(1, 1)
        wait_plain(0)
        start_add(0, 0)

        def ring_body(gg, carry):
            for b in range(NBUF):
                ci = NBUF * gg + b

                @pl.when(ci + 2 < NCH)
                def _():
                    start_plain(ci + 2, (b + 2) % NBUF)

                @pl.when(ci + 1 < NCH)
                def _():
                    wait_plain((b + 1) % NBUF)
                    start_add(ci + 1, (b + 1) % NBUF)

                wait_add(b)
                compute(ci, b)

            return carry

        lax.fori_loop(0, NCH // NBUF, ring_body, 0, unroll=False)

        for ci in range(NCH - NCH % NBUF, NCH):
            b = ci % NBUF

            @pl.when(ci + 2 < NCH)
            def _():
                start_plain(ci + 2, (b + 2) % NBUF)

            @pl.when(ci + 1 < NCH)
            def _():
                wait_plain((b + 1) % NBUF)
                start_add(ci + 1, (b + 1) % NBUF)

            wait_add(b)
            compute(ci, b)

        if TAIL:
            toff = NCH * CHUNK
            tb = rows.at[0].at[pl.ds(0, TAIL)]
            pltpu.async_copy(tab_hbm.at[idx0_all.at[pl.ds(toff, TAIL)]],
                             tb, semp[0])
            pltpu.make_async_copy(tab_hbm.at[pl.ds(0, TAIL)], tb,
                                  semp[0]).wait()
            pltpu.async_copy(tabn_hbm.at[idx1_all.at[pl.ds(toff, TAIL)]],
                             tb, sema[0], add=True)
            pltpu.make_async_copy(tabn_hbm.at[pl.ds(0, TAIL)], tb,
                                  sema[0]).wait()
            r = rows.at[0]
            for g in range(TAIL // L):
                out_all[pl.ds(toff + g * L, L)] = group_probs(r, g)

        pltpu.sync_copy(out_all, out_hbm.at[pl.ds(base, E_W)])

    return decode(tab, tabn, idx0, idx1)


def kernel(h, idx):
    idx = idx.astype(jnp.int32)
    idx0 = idx[:, 0]
    idx1 = idx[:, 1]
    tab, tabn = _renorm_tc(h)
    return _decode_sc(tab, tabn, idx0, idx1)


# speedup vs baseline: 1.1369x; 1.0154x over previous
import functools

import jax
import jax.numpy as jnp
from jax import lax
from jax.experimental import pallas as pl
from jax.experimental.pallas import tpu as pltpu
from jax.experimental.pallas import tpu_sc as plsc

R = 2.0
T = 1.0

NC = 2
NS = 16
NW = NC * NS
L = 16

N_NODES = 10000
D = 128
N_EDGES = 320000
CHUNK = 128
NBLK = N_EDGES // CHUNK
BPW = NBLK // NW
NXTRA = NBLK - BPW * NW
NBUF = 3


def _renorm_tc(h):
    blk = 1000

    def body(h_ref, o_ref, on_ref):
        x = h_ref[...]
        ss = jnp.sum(x * x, axis=1, keepdims=True)
        norm = jnp.sqrt(ss)
        scale = jnp.where(norm > 1.0, 1.0 / jnp.maximum(norm, 1e-12), 1.0)
        y = x * scale
        o_ref[...] = y
        on_ref[...] = -y

    return pl.pallas_call(
        body,
        out_shape=[jax.ShapeDtypeStruct((N_NODES, D), jnp.float32),
                   jax.ShapeDtypeStruct((N_NODES, D), jnp.float32)],
        grid=(N_NODES // blk,),
        in_specs=[pl.BlockSpec((blk, D), lambda i: (i, 0))],
        out_specs=[pl.BlockSpec((blk, D), lambda i: (i, 0)),
                   pl.BlockSpec((blk, D), lambda i: (i, 0))],
    )(h)


def _decode_sc(tab, tabn, idx_flat):
    mesh = plsc.VectorSubcoreMesh(core_axis_name="c", subcore_axis_name="s")

    @functools.partial(
        pl.kernel,
        out_type=jax.ShapeDtypeStruct((N_EDGES,), jnp.float32),
        mesh=mesh,
        scratch_types=[
            pltpu.VMEM((BPW * 2 * CHUNK,), jnp.int32),
            pltpu.VMEM((2 * CHUNK,), jnp.int32),
            pltpu.VMEM((NBUF, CHUNK, D), jnp.float32),
            pltpu.VMEM((BPW * CHUNK,), jnp.float32),
            pltpu.VMEM((CHUNK,), jnp.float32),
            pltpu.SemaphoreType.DMA,
            pltpu.SemaphoreType.DMA,
            pltpu.SemaphoreType.DMA,
            pltpu.SemaphoreType.DMA,
            pltpu.SemaphoreType.DMA,
            pltpu.SemaphoreType.DMA,
        ],
    )
    def decode(tab_hbm, tabn_hbm, idxf_hbm, out_hbm,
               idx_all, idx_tail, rows, out_all, out_tail,
               semp0, semp1, semp2, sema0, sema1, sema2):
        semp = [semp0, semp1, semp2]
        sema = [sema0, sema1, sema2]
        wid = lax.axis_index("s") * NC + lax.axis_index("c")
        blk0 = wid * BPW
        ebase = blk0 * CHUNK

        pltpu.sync_copy(idxf_hbm.at[pl.ds(blk0 * 2 * CHUNK, BPW * 2 * CHUNK)],
                        idx_all)

        def start_plain(ci, b):
            pltpu.async_copy(
                tab_hbm.at[idx_all.at[pl.ds(ci * 2 * CHUNK, CHUNK)]],
                rows.at[b], semp[b])

        def wait_plain(b):
            pltpu.make_async_copy(tab_hbm.at[pl.ds(0, CHUNK)],
                                  rows.at[b], semp[b]).wait()

        def start_add(ci, b):
            pltpu.async_copy(
                tabn_hbm.at[idx_all.at[pl.ds(ci * 2 * CHUNK + CHUNK, CHUNK)]],
                rows.at[b], sema[b], add=True)

        def wait_add(b):
            pltpu.make_async_copy(tabn_hbm.at[pl.ds(0, CHUNK)],
                                  rows.at[b], sema[b]).wait()

        lane = lax.iota(jnp.int32, L)
        dnums = lax.GatherDimensionNumbers(
            offset_dims=(), collapsed_slice_dims=(0,),
            start_index_map=(0,))

        def xperm(v, p):
            return lax.gather(v, p, dnums, slice_sizes=(1,),
                              mode=lax.GatherScatterMode.PROMISE_IN_BOUNDS)

        tree_perm = [(lane ^ (1 << k))[:, None] for k in range(4)]
        tree_mask = [(lane & (1 << k)) == 0 for k in range(4)]

        def merge_level(vs, k):
            p, m = tree_perm[k], tree_mask[k]
            return [jnp.where(m, x + xperm(x, p), y + xperm(y, p))
                    for x, y in zip(vs[0::2], vs[1::2])]

        def group_probs(r, g):
            quads = []
            vs = []
            for k in range(L):
                e = g * L + k
                acc = None
                for d in range(D // L):
                    df = r[e, pl.ds(d * L, L)]
                    sq = df * df
                    acc = sq if acc is None else acc + sq
                vs.append(acc)
                if len(vs) == 4:
                    quads.append(merge_level(merge_level(vs, 0), 1)[0])
                    vs = []
            res = merge_level(merge_level(quads, 2), 3)[0]
            return 1.0 / (jnp.exp((res - R) / T) + 1.0)

        def compute(ci, b):
            obase = ci * CHUNK
            r = rows.at[b]

            def group_body(g, c2):
                out_all[pl.ds(obase + g * L, L)] = group_probs(r, g)
                return c2

            lax.fori_loop(0, CHUNK // L, group_body, 0, unroll=False)

        start_plain(0, 0)
        start_plain(1, 1)
        wait_plain(0)
        start_add(0, 0)

        def ring_body(gg, carry):
            for b in range(NBUF):
                ci = NBUF * gg + b

                @pl.when(ci + 2 < BPW)
                def _():
                    start_plain(ci + 2, (b + 2) % NBUF)

                @pl.when(ci + 1 < BPW)
                def _():
                    wait_plain((b + 1) % NBUF)
                    start_add(ci + 1, (b + 1) % NBUF)

                wait_add(b)
                compute(ci, b)

            return carry

        lax.fori_loop(0, BPW // NBUF, ring_body, 0, unroll=False)

        pltpu.sync_copy(out_all, out_hbm.at[pl.ds(ebase, BPW * CHUNK)])

        @pl.when(wid < NXTRA)
        def _():
            tblk = NW * BPW + wid
            pltpu.sync_copy(idxf_hbm.at[pl.ds(tblk * 2 * CHUNK, 2 * CHUNK)],
                            idx_tail)
            tb = rows.at[0]
            pltpu.async_copy(tab_hbm.at[idx_tail.at[pl.ds(0, CHUNK)]],
                             tb, semp[0])
            pltpu.make_async_copy(tab_hbm.at[pl.ds(0, CHUNK)], tb,
                                  semp[0]).wait()
            pltpu.async_copy(tabn_hbm.at[idx_tail.at[pl.ds(CHUNK, CHUNK)]],
                             tb, sema[0], add=True)
            pltpu.make_async_copy(tabn_hbm.at[pl.ds(0, CHUNK)], tb,
                                  sema[0]).wait()

            def tail_group(g, c2):
                out_tail[pl.ds(g * L, L)] = group_probs(tb, g)
                return c2

            lax.fori_loop(0, CHUNK // L, tail_group, 0, unroll=False)
            pltpu.sync_copy(out_tail, out_hbm.at[pl.ds(tblk * CHUNK, CHUNK)])

    return decode(tab, tabn, idx_flat)


def kernel(h, idx):
    idx = idx.astype(jnp.int32)
    idx_flat = jnp.transpose(idx.reshape(NBLK, CHUNK, 2),
                             (0, 2, 1)).reshape(NBLK * 2 * CHUNK)
    tab, tabn = _renorm_tc(h)
    return _decode_sc(tab, tabn, idx_flat)
